# Initial kernel scaffold; baseline (speedup 1.0000x reference)
import jax
import jax.numpy as jnp
from jax.experimental import pallas as pl


def kernel(x, edge_index, W_self1, W_neigh1, b1, W_self2, W_neigh2, b2):
    raise NotImplementedError("write your pallas kernel here")


# speedup vs baseline: 4.9675x; 4.9675x over previous
import functools

import jax
import jax.numpy as jnp
from jax import lax
from jax.experimental import pallas as pl
from jax.experimental.pallas import tpu as pltpu
from jax.experimental.pallas import tpu_sc as plsc

N = 10000
E = 320000
D = 128

NC = 2
NS = 16
B = 80
EPT = E // NS
NB = EPT // B
NP = 10240

D1 = 72
D2 = 64

_ROWBLK = 1000
_GRID = N // _ROWBLK


def _make_sc_agg(d):
  mesh = plsc.VectorSubcoreMesh(
      core_axis_name="c", subcore_axis_name="s", num_cores=NC, num_subcores=NS)

  zoffs = sorted({j * 16 for j in range(d // 16)} | {d - 16})

  @functools.partial(
      pl.kernel,
      out_type=jax.ShapeDtypeStruct((NC, NP, d), jnp.float32),
      mesh=mesh,
      compiler_params=pltpu.CompilerParams(use_tc_tiling_on_sc=False),
      scratch_types=[
          pltpu.VMEM((NB, B), jnp.int32),
          pltpu.VMEM((NB, B), jnp.int32),
          pltpu.VMEM((B, d), jnp.float32),
          pltpu.VMEM((128, d), jnp.float32),
          pltpu.VMEM_SHARED((NP, d), jnp.float32),
      ],
  )
  def agg(hlo_hbm, hhi_hbm, src_hbm, dst_hbm, out_hbm,
          src_v, dst_v, rows_v, zbuf, acc):
    c = lax.axis_index("c")
    s = lax.axis_index("s")

    z16 = jnp.zeros((16,), jnp.float32)

    def zrow(i, carry):
      for o in zoffs:
        zbuf[i, pl.ds(o, 16)] = z16
      return carry

    lax.fori_loop(0, 128, zrow, 0)
    for j in range(5):
      pltpu.sync_copy(zbuf, acc.at[pl.ds(s * 640 + j * 128, 128)])
    plsc.subcore_barrier()

    pltpu.sync_copy(src_hbm.at[s], src_v)
    pltpu.sync_copy(dst_hbm.at[s], dst_v)

    def run(h_hbm):
      def body(i, carry):
        pltpu.sync_copy(h_hbm.at[src_v.at[i]], rows_v)
        pltpu.sync_copy(rows_v, acc.at[dst_v.at[i]], add=True)
        return carry

      lax.fori_loop(0, NB, body, 0)

    @pl.when(c == 0)
    def _():
      run(hlo_hbm)

    @pl.when(c == 1)
    def _():
      run(hhi_hbm)

    plsc.subcore_barrier()

    for j in range(5):
      r = s * 640 + j * 128
      pltpu.sync_copy(acc.at[pl.ds(r, 128)], out_hbm.at[c, pl.ds(r, 128)])

  return agg


_sc_agg1 = _make_sc_agg(D1)
_sc_agg2 = _make_sc_agg(D2)


def _tc_pre(x_ref, ws_ref, wn_ref, b_ref, xs_ref, xlo_ref, xhi_ref):
  x = x_ref[...]
  xs_ref[...] = jnp.dot(x, ws_ref[...], preferred_element_type=jnp.float32) + b_ref[...]
  xn = jnp.dot(x, wn_ref[...], preferred_element_type=jnp.float32)
  xlo_ref[...] = xn[:, :D1]
  ones = jnp.ones((_ROWBLK, 1), jnp.float32)
  pad = jnp.zeros((_ROWBLK, D1 - (D - D1) - 1), jnp.float32)
  xhi_ref[...] = jnp.concatenate([xn[:, D1:], ones, pad], axis=1)


def _tc_mid(xs_ref, p0_ref, p1_ref, ws_ref, wn_ref, b_ref,
            xs2_ref, xlo2_ref, xhi2_ref, rdeg_ref):
  p0 = p0_ref[...]
  p1 = p1_ref[...]
  deg = p1[:, D - D1:D - D1 + 1]
  rdeg = 1.0 / jnp.maximum(deg, 1.0)
  neigh = jnp.concatenate([p0, p1[:, :D - D1]], axis=1)
  h1 = jnp.maximum(xs_ref[...] + neigh * rdeg, 0.0)
  xs2_ref[...] = jnp.dot(h1, ws_ref[...], preferred_element_type=jnp.float32) + b_ref[...]
  xn2 = jnp.dot(h1, wn_ref[...], preferred_element_type=jnp.float32)
  xlo2_ref[...] = xn2[:, :D2]
  xhi2_ref[...] = xn2[:, D2:]
  rdeg_ref[...] = rdeg


def _tc_post(xs2_ref, q0_ref, q1_ref, rdeg_ref, out_ref):
  neigh = jnp.concatenate([q0_ref[...], q1_ref[...]], axis=1)
  out_ref[...] = xs2_ref[...] + neigh * rdeg_ref[...]


def _row_spec(cols):
  return pl.BlockSpec((_ROWBLK, cols), lambda i: (i, 0))


_W_SPEC = pl.BlockSpec((D, D), lambda i: (0, 0))
_B_SPEC = pl.BlockSpec((1, D), lambda i: (0, 0))


def kernel(x, edge_index, W_self1, W_neigh1, b1, W_self2, W_neigh2, b2):
  src = edge_index[0].reshape(NS, NB, B)
  dst = edge_index[1].reshape(NS, NB, B)
  b1r = b1.reshape(1, D)
  b2r = b2.reshape(1, D)

  xs1, xlo1, xhi1 = pl.pallas_call(
      _tc_pre,
      grid=(_GRID,),
      in_specs=[_row_spec(D), _W_SPEC, _W_SPEC, _B_SPEC],
      out_specs=[_row_spec(D), _row_spec(D1), _row_spec(D1)],
      out_shape=[
          jax.ShapeDtypeStruct((N, D), jnp.float32),
          jax.ShapeDtypeStruct((N, D1), jnp.float32),
          jax.ShapeDtypeStruct((N, D1), jnp.float32),
      ],
  )(x, W_self1, W_neigh1, b1r)

  p = _sc_agg1(xlo1, xhi1, src, dst)

  xs2, xlo2, xhi2, rdeg = pl.pallas_call(
      _tc_mid,
      grid=(_GRID,),
      in_specs=[_row_spec(D), _row_spec(D1), _row_spec(D1), _W_SPEC, _W_SPEC,
                _B_SPEC],
      out_specs=[_row_spec(D), _row_spec(D2), _row_spec(D2), _row_spec(1)],
      out_shape=[
          jax.ShapeDtypeStruct((N, D), jnp.float32),
          jax.ShapeDtypeStruct((N, D2), jnp.float32),
          jax.ShapeDtypeStruct((N, D2), jnp.float32),
          jax.ShapeDtypeStruct((N, 1), jnp.float32),
      ],
  )(xs1, p[0], p[1], W_self2, W_neigh2, b2r)

  q = _sc_agg2(xlo2, xhi2, src, dst)

  out = pl.pallas_call(
      _tc_post,
      grid=(_GRID,),
      in_specs=[_row_spec(D), _row_spec(D2), _row_spec(D2), _row_spec(1)],
      out_specs=_row_spec(D),
      out_shape=jax.ShapeDtypeStruct((N, D), jnp.float32),
  )(xs2, q[0], q[1], rdeg)

  return out


# speedup vs baseline: 11.1154x; 2.2376x over previous
import functools

import jax
import jax.numpy as jnp
from jax import lax
from jax.experimental import pallas as pl
from jax.experimental.pallas import tpu as pltpu
from jax.experimental.pallas import tpu_sc as plsc

N = 10000
E = 320000
D = 128

NC = 2
NS = 16
B = 80
EPT = E // NS
NB = EPT // B
NP = 10240
R = 5

D1 = 72
D2 = 64

_ROWBLK = 1000
_GRID = N // _ROWBLK


def _make_sc_agg(d):
  mesh = plsc.VectorSubcoreMesh(
      core_axis_name="c", subcore_axis_name="s", num_cores=NC, num_subcores=NS)

  zoffs = sorted({j * 16 for j in range(d // 16)} | {d - 16})

  @functools.partial(
      pl.kernel,
      out_type=jax.ShapeDtypeStruct((NC, NP, d), jnp.float32),
      mesh=mesh,
      compiler_params=pltpu.CompilerParams(use_tc_tiling_on_sc=False),
      scratch_types=[
          pltpu.VMEM((NB, B), jnp.int32),
          pltpu.VMEM((NB, B), jnp.int32),
          [pltpu.VMEM((B, d), jnp.float32) for _ in range(R)],
          [pltpu.SemaphoreType.DMA for _ in range(R)],
          pltpu.VMEM((128, d), jnp.float32),
          pltpu.VMEM_SHARED((NP, d), jnp.float32),
      ],
  )
  def agg(hlo_hbm, hhi_hbm, src_hbm, dst_hbm, out_hbm,
          src_v, dst_v, rows, gsems, zbuf, acc):
    c = lax.axis_index("c")
    s = lax.axis_index("s")

    z16 = jnp.zeros((16,), jnp.float32)

    def zrow(i, carry):
      for o in zoffs:
        zbuf[i, pl.ds(o, 16)] = z16
      return carry

    lax.fori_loop(0, 128, zrow, 0)
    for j in range(5):
      pltpu.sync_copy(zbuf, acc.at[pl.ds(s * 640 + j * 128, 128)])
    plsc.subcore_barrier()

    pltpu.sync_copy(src_hbm.at[s], src_v)
    pltpu.sync_copy(dst_hbm.at[s], dst_v)

    def run(h_hbm):
      for r in range(R):
        pltpu.async_copy(h_hbm.at[src_v.at[r]], rows[r], gsems[r])

      def body(g, carry):
        for r in range(R):
          i = R * g + r
          pltpu.make_async_copy(h_hbm.at[src_v.at[i]], rows[r],
                                gsems[r]).wait()
          pltpu.sync_copy(rows[r], acc.at[dst_v.at[i]], add=True)

          @pl.when(i + R < NB)
          def _():
            pltpu.async_copy(h_hbm.at[src_v.at[i + R]], rows[r], gsems[r])

        return carry

      lax.fori_loop(0, NB // R, body, 0)

    @pl.when(c == 0)
    def _():
      run(hlo_hbm)

    @pl.when(c == 1)
    def _():
      run(hhi_hbm)

    plsc.subcore_barrier()

    for j in range(5):
      r = s * 640 + j * 128
      pltpu.sync_copy(acc.at[pl.ds(r, 128)], out_hbm.at[c, pl.ds(r, 128)])

  return agg


_sc_agg1 = _make_sc_agg(D1)
_sc_agg2 = _make_sc_agg(D2)


def _tc_pre(x_ref, ws_ref, wn_ref, b_ref, xs_ref, xlo_ref, xhi_ref):
  x = x_ref[...]
  xs_ref[...] = jnp.dot(x, ws_ref[...], preferred_element_type=jnp.float32) + b_ref[...]
  xn = jnp.dot(x, wn_ref[...], preferred_element_type=jnp.float32)
  xlo_ref[...] = xn[:, :D1]
  ones = jnp.ones((_ROWBLK, 1), jnp.float32)
  pad = jnp.zeros((_ROWBLK, D1 - (D - D1) - 1), jnp.float32)
  xhi_ref[...] = jnp.concatenate([xn[:, D1:], ones, pad], axis=1)


def _tc_mid(xs_ref, p0_ref, p1_ref, ws_ref, wn_ref, b_ref,
            xs2_ref, xlo2_ref, xhi2_ref, rdeg_ref):
  p0 = p0_ref[...]
  p1 = p1_ref[...]
  deg = p1[:, D - D1:D - D1 + 1]
  rdeg = 1.0 / jnp.maximum(deg, 1.0)
  neigh = jnp.concatenate([p0, p1[:, :D - D1]], axis=1)
  h1 = jnp.maximum(xs_ref[...] + neigh * rdeg, 0.0)
  xs2_ref[...] = jnp.dot(h1, ws_ref[...], preferred_element_type=jnp.float32) + b_ref[...]
  xn2 = jnp.dot(h1, wn_ref[...], preferred_element_type=jnp.float32)
  xlo2_ref[...] = xn2[:, :D2]
  xhi2_ref[...] = xn2[:, D2:]
  rdeg_ref[...] = rdeg


def _tc_post(xs2_ref, q0_ref, q1_ref, rdeg_ref, out_ref):
  neigh = jnp.concatenate([q0_ref[...], q1_ref[...]], axis=1)
  out_ref[...] = xs2_ref[...] + neigh * rdeg_ref[...]


def _row_spec(cols):
  return pl.BlockSpec((_ROWBLK, cols), lambda i: (i, 0))


_W_SPEC = pl.BlockSpec((D, D), lambda i: (0, 0))
_B_SPEC = pl.BlockSpec((1, D), lambda i: (0, 0))


def kernel(x, edge_index, W_self1, W_neigh1, b1, W_self2, W_neigh2, b2):
  src = edge_index[0].reshape(NS, NB, B)
  dst = edge_index[1].reshape(NS, NB, B)
  b1r = b1.reshape(1, D)
  b2r = b2.reshape(1, D)

  xs1, xlo1, xhi1 = pl.pallas_call(
      _tc_pre,
      grid=(_GRID,),
      in_specs=[_row_spec(D), _W_SPEC, _W_SPEC, _B_SPEC],
      out_specs=[_row_spec(D), _row_spec(D1), _row_spec(D1)],
      out_shape=[
          jax.ShapeDtypeStruct((N, D), jnp.float32),
          jax.ShapeDtypeStruct((N, D1), jnp.float32),
          jax.ShapeDtypeStruct((N, D1), jnp.float32),
      ],
  )(x, W_self1, W_neigh1, b1r)

  p = _sc_agg1(xlo1, xhi1, src, dst)

  xs2, xlo2, xhi2, rdeg = pl.pallas_call(
      _tc_mid,
      grid=(_GRID,),
      in_specs=[_row_spec(D), _row_spec(D1), _row_spec(D1), _W_SPEC, _W_SPEC,
                _B_SPEC],
      out_specs=[_row_spec(D), _row_spec(D2), _row_spec(D2), _row_spec(1)],
      out_shape=[
          jax.ShapeDtypeStruct((N, D), jnp.float32),
          jax.ShapeDtypeStruct((N, D2), jnp.float32),
          jax.ShapeDtypeStruct((N, D2), jnp.float32),
          jax.ShapeDtypeStruct((N, 1), jnp.float32),
      ],
  )(xs1, p[0], p[1], W_self2, W_neigh2, b2r)

  q = _sc_agg2(xlo2, xhi2, src, dst)

  out = pl.pallas_call(
      _tc_post,
      grid=(_GRID,),
      in_specs=[_row_spec(D), _row_spec(D2), _row_spec(D2), _row_spec(1)],
      out_specs=_row_spec(D),
      out_shape=jax.ShapeDtypeStruct((N, D), jnp.float32),
  )(xs2, q[0], q[1], rdeg)

  return out


# speedup vs baseline: 12.1423x; 1.0924x over previous
import functools

import jax
import jax.numpy as jnp
from jax import lax
from jax.experimental import pallas as pl
from jax.experimental.pallas import tpu as pltpu
from jax.experimental.pallas import tpu_sc as plsc

N = 10000
E = 320000
D = 128

NC = 2
NS = 16
B = 80
EPT = E // NS
NB = EPT // B
NP = 10240
R = 5

D1 = 72
D2 = 64

_ROWBLK = 1000
_GRID = N // _ROWBLK


def _make_sc_agg(d):
  mesh = plsc.VectorSubcoreMesh(
      core_axis_name="c", subcore_axis_name="s", num_cores=NC, num_subcores=NS)

  zoffs = sorted({j * 16 for j in range(d // 16)} | {d - 16})

  @functools.partial(
      pl.kernel,
      out_type=jax.ShapeDtypeStruct((NC, NP, d), jnp.float32),
      mesh=mesh,
      compiler_params=pltpu.CompilerParams(use_tc_tiling_on_sc=False),
      scratch_types=[
          pltpu.VMEM((NB, B), jnp.int32),
          pltpu.VMEM((NB, B), jnp.int32),
          [pltpu.VMEM((B, d), jnp.float32) for _ in range(R)],
          [pltpu.SemaphoreType.DMA for _ in range(R)],
          pltpu.VMEM((128, d), jnp.float32),
          pltpu.VMEM_SHARED((NP, d), jnp.float32),
      ],
  )
  def agg(hlo_hbm, hhi_hbm, eidx_hbm, out_hbm,
          src_v, dst_v, rows, gsems, zbuf, acc):
    c = lax.axis_index("c")
    s = lax.axis_index("s")

    z16 = jnp.zeros((16,), jnp.float32)

    def zrow(i, carry):
      for o in zoffs:
        zbuf[i, pl.ds(o, 16)] = z16
      return carry

    lax.fori_loop(0, 128, zrow, 0)
    for j in range(5):
      pltpu.sync_copy(zbuf, acc.at[pl.ds(s * 640 + j * 128, 128)])
    plsc.subcore_barrier()

    pltpu.sync_copy(eidx_hbm.at[0, s], src_v)
    pltpu.sync_copy(eidx_hbm.at[1, s], dst_v)

    def run(h_hbm):
      for r in range(R):
        pltpu.async_copy(h_hbm.at[src_v.at[r]], rows[r], gsems[r])

      def body(g, carry):
        for r in range(R):
          i = R * g + r
          pltpu.make_async_copy(h_hbm.at[src_v.at[i]], rows[r],
                                gsems[r]).wait()
          pltpu.sync_copy(rows[r], acc.at[dst_v.at[i]], add=True)

          @pl.when(i + R < NB)
          def _():
            pltpu.async_copy(h_hbm.at[src_v.at[i + R]], rows[r], gsems[r])

        return carry

      lax.fori_loop(0, NB // R, body, 0)

    @pl.when(c == 0)
    def _():
      run(hlo_hbm)

    @pl.when(c == 1)
    def _():
      run(hhi_hbm)

    plsc.subcore_barrier()

    for j in range(5):
      r = s * 640 + j * 128
      pltpu.sync_copy(acc.at[pl.ds(r, 128)], out_hbm.at[c, pl.ds(r, 128)])

  return agg


_sc_agg1 = _make_sc_agg(D1)
_sc_agg2 = _make_sc_agg(D2)


def _tc_pre(x_ref, ws_ref, wn_ref, b_ref, xs_ref, xlo_ref, xhi_ref):
  x = x_ref[...]
  xs_ref[...] = jnp.dot(x, ws_ref[...], preferred_element_type=jnp.float32) + b_ref[...]
  xn = jnp.dot(x, wn_ref[...], preferred_element_type=jnp.float32)
  xlo_ref[...] = xn[:, :D1]
  ones = jnp.ones((_ROWBLK, 1), jnp.float32)
  pad = jnp.zeros((_ROWBLK, D1 - (D - D1) - 1), jnp.float32)
  xhi_ref[...] = jnp.concatenate([xn[:, D1:], ones, pad], axis=1)


def _tc_mid(xs_ref, p0_ref, p1_ref, ws_ref, wn_ref, b_ref,
            xs2_ref, xlo2_ref, xhi2_ref, rdeg_ref):
  p0 = p0_ref[0]
  p1 = p1_ref[0]
  deg = p1[:, D - D1:D - D1 + 1]
  rdeg = 1.0 / jnp.maximum(deg, 1.0)
  neigh = jnp.concatenate([p0, p1[:, :D - D1]], axis=1)
  h1 = jnp.maximum(xs_ref[...] + neigh * rdeg, 0.0)
  xs2_ref[...] = jnp.dot(h1, ws_ref[...], preferred_element_type=jnp.float32) + b_ref[...]
  xn2 = jnp.dot(h1, wn_ref[...], preferred_element_type=jnp.float32)
  xlo2_ref[...] = xn2[:, :D2]
  xhi2_ref[...] = xn2[:, D2:]
  rdeg_ref[...] = rdeg


def _tc_post(xs2_ref, q0_ref, q1_ref, rdeg_ref, out_ref):
  neigh = jnp.concatenate([q0_ref[0], q1_ref[0]], axis=1)
  out_ref[...] = xs2_ref[...] + neigh * rdeg_ref[...]


def _row_spec(cols):
  return pl.BlockSpec((_ROWBLK, cols), lambda i: (i, 0))


_W_SPEC = pl.BlockSpec((D, D), lambda i: (0, 0))
_B_SPEC = pl.BlockSpec((1, D), lambda i: (0, 0))


def _part_spec(cols, core):
  return pl.BlockSpec((1, _ROWBLK, cols), lambda i, c=core: (c, i, 0))


def kernel(x, edge_index, W_self1, W_neigh1, b1, W_self2, W_neigh2, b2):
  eidx = edge_index.reshape(2, NS, NB, B)
  b1r = b1.reshape(1, D)
  b2r = b2.reshape(1, D)

  xs1, xlo1, xhi1 = pl.pallas_call(
      _tc_pre,
      grid=(_GRID,),
      in_specs=[_row_spec(D), _W_SPEC, _W_SPEC, _B_SPEC],
      out_specs=[_row_spec(D), _row_spec(D1), _row_spec(D1)],
      out_shape=[
          jax.ShapeDtypeStruct((N, D), jnp.float32),
          jax.ShapeDtypeStruct((N, D1), jnp.float32),
          jax.ShapeDtypeStruct((N, D1), jnp.float32),
      ],
  )(x, W_self1, W_neigh1, b1r)

  p = _sc_agg1(xlo1, xhi1, eidx)

  xs2, xlo2, xhi2, rdeg = pl.pallas_call(
      _tc_mid,
      grid=(_GRID,),
      in_specs=[_row_spec(D), _part_spec(D1, 0), _part_spec(D1, 1), _W_SPEC,
                _W_SPEC, _B_SPEC],
      out_specs=[_row_spec(D), _row_spec(D2), _row_spec(D2), _row_spec(1)],
      out_shape=[
          jax.ShapeDtypeStruct((N, D), jnp.float32),
          jax.ShapeDtypeStruct((N, D2), jnp.float32),
          jax.ShapeDtypeStruct((N, D2), jnp.float32),
          jax.ShapeDtypeStruct((N, 1), jnp.float32),
      ],
  )(xs1, p, p, W_self2, W_neigh2, b2r)

  q = _sc_agg2(xlo2, xhi2, eidx)

  out = pl.pallas_call(
      _tc_post,
      grid=(_GRID,),
      in_specs=[_row_spec(D), _part_spec(D2, 0), _part_spec(D2, 1),
                _row_spec(1)],
      out_specs=_row_spec(D),
      out_shape=jax.ShapeDtypeStruct((N, D), jnp.float32),
  )(xs2, q, q, rdeg)

  return out


# speedup vs baseline: 12.1630x; 1.0017x over previous
import functools

import jax
import jax.numpy as jnp
from jax import lax
from jax.experimental import pallas as pl
from jax.experimental.pallas import tpu as pltpu
from jax.experimental.pallas import tpu_sc as plsc

N = 10000
E = 320000
D = 128

NC = 2
NS = 16
B = 80
EPT = E // NS
NB = EPT // B
NP = 10240
S = 6
F = 4

D1 = 72
D2 = 64

_ROWBLK = 1000
_GRID = N // _ROWBLK


def _make_sc_agg(d):
  mesh = plsc.VectorSubcoreMesh(
      core_axis_name="c", subcore_axis_name="s", num_cores=NC, num_subcores=NS)

  zoffs = sorted({j * 16 for j in range(d // 16)} | {d - 16})

  @functools.partial(
      pl.kernel,
      out_type=jax.ShapeDtypeStruct((NC, NP, d), jnp.float32),
      mesh=mesh,
      compiler_params=pltpu.CompilerParams(use_tc_tiling_on_sc=False),
      scratch_types=[
          pltpu.VMEM((NB, B), jnp.int32),
          pltpu.VMEM((NB, B), jnp.int32),
          [pltpu.VMEM((B, d), jnp.float32) for _ in range(S)],
          [pltpu.SemaphoreType.DMA for _ in range(S)],
          [pltpu.SemaphoreType.DMA for _ in range(S)],
          pltpu.VMEM((128, d), jnp.float32),
          pltpu.VMEM_SHARED((NP, d), jnp.float32),
      ],
  )
  def agg(hlo_hbm, hhi_hbm, eidx_hbm, out_hbm,
          src_v, dst_v, rows, gsems, ssems, zbuf, acc):
    c = lax.axis_index("c")
    s = lax.axis_index("s")

    z16 = jnp.zeros((16,), jnp.float32)

    def zrow(i, carry):
      for o in zoffs:
        zbuf[i, pl.ds(o, 16)] = z16
      return carry

    lax.fori_loop(0, 128, zrow, 0)
    for j in range(5):
      pltpu.sync_copy(zbuf, acc.at[pl.ds(s * 640 + j * 128, 128)])
    plsc.subcore_barrier()

    pltpu.sync_copy(eidx_hbm.at[0, s], src_v)
    pltpu.sync_copy(eidx_hbm.at[1, s], dst_v)

    def run(h_hbm):
      def visit(i, r, guard_ssem):
        r2 = (r + F) % S
        wait_sc = lambda: pltpu.make_async_copy(
            rows[r2], acc.at[dst_v.at[i + F - S]], ssems[r2]).wait()
        if guard_ssem:
          pl.when(i + F - S >= 0)(wait_sc)
        else:
          wait_sc()
        pltpu.async_copy(h_hbm.at[src_v.at[i + F]], rows[r2], gsems[r2])
        pltpu.make_async_copy(h_hbm.at[src_v.at[i]], rows[r], gsems[r]).wait()
        pltpu.async_copy(rows[r], acc.at[dst_v.at[i]], ssems[r], add=True)

      for j in range(F):
        pltpu.async_copy(h_hbm.at[src_v.at[j]], rows[j], gsems[j])

      nloop = (NB - F) // S

      def body(g, carry):
        for r in range(S):
          visit(S * g + r, r, guard_ssem=True)
        return carry

      lax.fori_loop(0, nloop, body, 0)

      for i in range(S * nloop, NB):
        r = i % S
        pltpu.make_async_copy(h_hbm.at[src_v.at[i]], rows[r], gsems[r]).wait()
        pltpu.async_copy(rows[r], acc.at[dst_v.at[i]], ssems[r], add=True)

      for j in range(S * nloop + F - S, NB):
        rj = j % S
        pltpu.make_async_copy(rows[rj], acc.at[dst_v.at[j]], ssems[rj]).wait()

    @pl.when(c == 0)
    def _():
      run(hlo_hbm)

    @pl.when(c == 1)
    def _():
      run(hhi_hbm)

    plsc.subcore_barrier()

    for j in range(5):
      r = s * 640 + j * 128
      pltpu.sync_copy(acc.at[pl.ds(r, 128)], out_hbm.at[c, pl.ds(r, 128)])

  return agg


_sc_agg1 = _make_sc_agg(D1)
_sc_agg2 = _make_sc_agg(D2)


def _tc_pre(x_ref, ws_ref, wn_ref, b_ref, xs_ref, xlo_ref, xhi_ref):
  x = x_ref[...]
  xs_ref[...] = jnp.dot(x, ws_ref[...], preferred_element_type=jnp.float32) + b_ref[...]
  xn = jnp.dot(x, wn_ref[...], preferred_element_type=jnp.float32)
  xlo_ref[...] = xn[:, :D1]
  ones = jnp.ones((_ROWBLK, 1), jnp.float32)
  pad = jnp.zeros((_ROWBLK, D1 - (D - D1) - 1), jnp.float32)
  xhi_ref[...] = jnp.concatenate([xn[:, D1:], ones, pad], axis=1)


def _tc_mid(xs_ref, p0_ref, p1_ref, ws_ref, wn_ref, b_ref,
            xs2_ref, xlo2_ref, xhi2_ref, rdeg_ref):
  p0 = p0_ref[0]
  p1 = p1_ref[0]
  deg = p1[:, D - D1:D - D1 + 1]
  rdeg = 1.0 / jnp.maximum(deg, 1.0)
  neigh = jnp.concatenate([p0, p1[:, :D - D1]], axis=1)
  h1 = jnp.maximum(xs_ref[...] + neigh * rdeg, 0.0)
  xs2_ref[...] = jnp.dot(h1, ws_ref[...], preferred_element_type=jnp.float32) + b_ref[...]
  xn2 = jnp.dot(h1, wn_ref[...], preferred_element_type=jnp.float32)
  xlo2_ref[...] = xn2[:, :D2]
  xhi2_ref[...] = xn2[:, D2:]
  rdeg_ref[...] = rdeg


def _tc_post(xs2_ref, q0_ref, q1_ref, rdeg_ref, out_ref):
  neigh = jnp.concatenate([q0_ref[0], q1_ref[0]], axis=1)
  out_ref[...] = xs2_ref[...] + neigh * rdeg_ref[...]


def _row_spec(cols):
  return pl.BlockSpec((_ROWBLK, cols), lambda i: (i, 0))


_W_SPEC = pl.BlockSpec((D, D), lambda i: (0, 0))
_B_SPEC = pl.BlockSpec((1, D), lambda i: (0, 0))


def _part_spec(cols, core):
  return pl.BlockSpec((1, _ROWBLK, cols), lambda i, c=core: (c, i, 0))


def kernel(x, edge_index, W_self1, W_neigh1, b1, W_self2, W_neigh2, b2):
  eidx = edge_index.reshape(2, NS, NB, B)
  b1r = b1.reshape(1, D)
  b2r = b2.reshape(1, D)

  xs1, xlo1, xhi1 = pl.pallas_call(
      _tc_pre,
      grid=(_GRID,),
      in_specs=[_row_spec(D), _W_SPEC, _W_SPEC, _B_SPEC],
      out_specs=[_row_spec(D), _row_spec(D1), _row_spec(D1)],
      out_shape=[
          jax.ShapeDtypeStruct((N, D), jnp.float32),
          jax.ShapeDtypeStruct((N, D1), jnp.float32),
          jax.ShapeDtypeStruct((N, D1), jnp.float32),
      ],
  )(x, W_self1, W_neigh1, b1r)

  p = _sc_agg1(xlo1, xhi1, eidx)

  xs2, xlo2, xhi2, rdeg = pl.pallas_call(
      _tc_mid,
      grid=(_GRID,),
      in_specs=[_row_spec(D), _part_spec(D1, 0), _part_spec(D1, 1), _W_SPEC,
                _W_SPEC, _B_SPEC],
      out_specs=[_row_spec(D), _row_spec(D2), _row_spec(D2), _row_spec(1)],
      out_shape=[
          jax.ShapeDtypeStruct((N, D), jnp.float32),
          jax.ShapeDtypeStruct((N, D2), jnp.float32),
          jax.ShapeDtypeStruct((N, D2), jnp.float32),
          jax.ShapeDtypeStruct((N, 1), jnp.float32),
      ],
  )(xs1, p, p, W_self2, W_neigh2, b2r)

  q = _sc_agg2(xlo2, xhi2, eidx)

  out = pl.pallas_call(
      _tc_post,
      grid=(_GRID,),
      in_specs=[_row_spec(D), _part_spec(D2, 0), _part_spec(D2, 1),
                _row_spec(1)],
      out_specs=_row_spec(D),
      out_shape=jax.ShapeDtypeStruct((N, D), jnp.float32),
  )(xs2, q, q, rdeg)

  return out
